# Initial kernel scaffold; baseline (speedup 1.0000x reference)
import jax
import jax.numpy as jnp
from jax.experimental import pallas as pl


def kernel(x, weight, bias):
    raise NotImplementedError("write your pallas kernel here")


# speedup vs baseline: 1.0128x; 1.0128x over previous
import jax
import jax.numpy as jnp
from jax.experimental import pallas as pl
from jax.experimental.pallas import tpu as pltpu


def _matmul_body(x_ref, wt_ref, b_ref, o_ref):
    xb = x_ref[...].astype(jnp.bfloat16)
    acc = jnp.dot(xb, wt_ref[...], preferred_element_type=jnp.float32)
    o_ref[...] = acc + b_ref[...]


def kernel(x, weight, bias):
    M, K = x.shape
    N = weight.shape[0]
    wt = weight.astype(jnp.bfloat16).T
    b2 = bias.reshape(1, N)

    tm = min(512, M)
    grid = (pl.cdiv(M, tm),)
    return pl.pallas_call(
        _matmul_body,
        out_shape=jax.ShapeDtypeStruct((M, N), x.dtype),
        grid=grid,
        in_specs=[
            pl.BlockSpec((tm, K), lambda i: (i, 0)),
            pl.BlockSpec((K, N), lambda i: (0, 0)),
            pl.BlockSpec((1, N), lambda i: (0, 0)),
        ],
        out_specs=pl.BlockSpec((tm, N), lambda i: (i, 0)),
        compiler_params=pltpu.CompilerParams(
            dimension_semantics=("parallel",),
            vmem_limit_bytes=32 * 1024 * 1024,
        ),
    )(x, wt, b2)


# speedup vs baseline: 1.0236x; 1.0107x over previous
import functools

import jax
import jax.numpy as jnp
from jax.experimental import pallas as pl
from jax.experimental.pallas import tpu as pltpu


def _matmul_body(x_ref, w_ref, b_ref, o_ref, wb_ref):
    @pl.when(pl.program_id(1) == 0)
    def _cast_weight():
        wb_ref[...] = w_ref[...].astype(jnp.bfloat16)

    xb = x_ref[...].astype(jnp.bfloat16)
    acc = jax.lax.dot_general(
        xb, wb_ref[...],
        dimension_numbers=(((1,), (1,)), ((), ())),
        preferred_element_type=jnp.float32,
    )
    o_ref[...] = acc + b_ref[...]


def kernel(x, weight, bias):
    M, K = x.shape
    N = weight.shape[0]
    b2 = bias.reshape(1, N)

    tm = min(512, M)
    blocks = pl.cdiv(M, tm)
    cores = 2 if blocks % 2 == 0 else 1
    inner = blocks // cores
    grid = (cores, inner)
    return pl.pallas_call(
        functools.partial(_matmul_body),
        out_shape=jax.ShapeDtypeStruct((M, N), x.dtype),
        grid=grid,
        in_specs=[
            pl.BlockSpec((tm, K), lambda i, j, inner=inner: (i * inner + j, 0)),
            pl.BlockSpec((N, K), lambda i, j: (0, 0)),
            pl.BlockSpec((1, N), lambda i, j: (0, 0)),
        ],
        out_specs=pl.BlockSpec((tm, N), lambda i, j, inner=inner: (i * inner + j, 0)),
        scratch_shapes=[pltpu.VMEM((N, K), jnp.bfloat16)],
        compiler_params=pltpu.CompilerParams(
            dimension_semantics=("parallel", "arbitrary"),
            vmem_limit_bytes=48 * 1024 * 1024,
        ),
    )(x, weight, b2)


# speedup vs baseline: 1.1780x; 1.1508x over previous
import functools

import jax
import jax.numpy as jnp
from jax.experimental import pallas as pl
from jax.experimental.pallas import tpu as pltpu


def _matmul_body(x_ref, w_ref, b_ref, o_ref, wb_ref):
    @pl.when(pl.program_id(1) == 0)
    def _cast_weight():
        wb_ref[...] = w_ref[...].astype(jnp.bfloat16)

    xb = x_ref[...].astype(jnp.bfloat16)
    acc = jax.lax.dot_general(
        xb, wb_ref[...],
        dimension_numbers=(((1,), (1,)), ((), ())),
        preferred_element_type=jnp.float32,
    )
    o_ref[...] = acc + b_ref[...]


def kernel(x, weight, bias):
    M, K = x.shape
    N = weight.shape[0]
    b2 = bias.reshape(1, N)

    tm = min(1024, M)
    blocks = pl.cdiv(M, tm)
    cores = 2 if blocks % 2 == 0 else 1
    inner = blocks // cores
    grid = (cores, inner)
    return pl.pallas_call(
        functools.partial(_matmul_body),
        out_shape=jax.ShapeDtypeStruct((M, N), x.dtype),
        grid=grid,
        in_specs=[
            pl.BlockSpec((tm, K), lambda i, j, inner=inner: (i * inner + j, 0)),
            pl.BlockSpec((N, K), lambda i, j: (0, 0)),
            pl.BlockSpec((1, N), lambda i, j: (0, 0)),
        ],
        out_specs=pl.BlockSpec((tm, N), lambda i, j, inner=inner: (i * inner + j, 0)),
        scratch_shapes=[pltpu.VMEM((N, K), jnp.bfloat16)],
        compiler_params=pltpu.CompilerParams(
            dimension_semantics=("parallel", "arbitrary"),
            vmem_limit_bytes=48 * 1024 * 1024,
        ),
    )(x, weight, b2)


# speedup vs baseline: 1.1847x; 1.0057x over previous
import functools

import jax
import jax.numpy as jnp
from jax.experimental import pallas as pl
from jax.experimental.pallas import tpu as pltpu


def _matmul_body(x_ref, w_ref, b_ref, o_ref, wb_ref):
    @pl.when(pl.program_id(1) == 0)
    def _cast_weight():
        wb_ref[...] = w_ref[...].astype(jnp.bfloat16)

    xb = x_ref[...].astype(jnp.bfloat16)
    acc = jax.lax.dot_general(
        xb, wb_ref[...],
        dimension_numbers=(((1,), (1,)), ((), ())),
        preferred_element_type=jnp.float32,
    )
    o_ref[...] = acc + b_ref[...]


def kernel(x, weight, bias):
    M, K = x.shape
    N = weight.shape[0]
    b2 = bias.reshape(1, N)

    tm = min(2048, M)
    blocks = pl.cdiv(M, tm)
    cores = 2 if blocks % 2 == 0 else 1
    inner = blocks // cores
    grid = (cores, inner)
    return pl.pallas_call(
        functools.partial(_matmul_body),
        out_shape=jax.ShapeDtypeStruct((M, N), x.dtype),
        grid=grid,
        in_specs=[
            pl.BlockSpec((tm, K), lambda i, j, inner=inner: (i * inner + j, 0)),
            pl.BlockSpec((N, K), lambda i, j: (0, 0)),
            pl.BlockSpec((1, N), lambda i, j: (0, 0)),
        ],
        out_specs=pl.BlockSpec((tm, N), lambda i, j, inner=inner: (i * inner + j, 0)),
        scratch_shapes=[pltpu.VMEM((N, K), jnp.bfloat16)],
        compiler_params=pltpu.CompilerParams(
            dimension_semantics=("parallel", "arbitrary"),
            vmem_limit_bytes=48 * 1024 * 1024,
        ),
    )(x, weight, b2)


# speedup vs baseline: 1.2032x; 1.0156x over previous
import jax
import jax.numpy as jnp
from jax.experimental import pallas as pl
from jax.experimental.pallas import tpu as pltpu


def _matmul_body(x_ref, w_ref, b_ref, o_ref):
    acc = jax.lax.dot_general(
        x_ref[...], w_ref[...],
        dimension_numbers=(((1,), (1,)), ((), ())),
        preferred_element_type=jnp.float32,
    )
    o_ref[...] = acc + b_ref[...]


def kernel(x, weight, bias):
    M, K = x.shape
    N = weight.shape[0]
    b2 = bias.reshape(1, N)

    tm = min(2048, M)
    blocks = pl.cdiv(M, tm)
    cores = 2 if blocks % 2 == 0 else 1
    inner = blocks // cores
    grid = (cores, inner)
    return pl.pallas_call(
        _matmul_body,
        out_shape=jax.ShapeDtypeStruct((M, N), x.dtype),
        grid=grid,
        in_specs=[
            pl.BlockSpec((tm, K), lambda i, j, inner=inner: (i * inner + j, 0)),
            pl.BlockSpec((N, K), lambda i, j: (0, 0)),
            pl.BlockSpec((1, N), lambda i, j: (0, 0)),
        ],
        out_specs=pl.BlockSpec((tm, N), lambda i, j, inner=inner: (i * inner + j, 0)),
        compiler_params=pltpu.CompilerParams(
            dimension_semantics=("parallel", "arbitrary"),
            vmem_limit_bytes=48 * 1024 * 1024,
        ),
    )(x, weight, b2)
